# baseline (speedup 1.0000x reference)
import functools

import jax
import jax.numpy as jnp
from jax import lax
from jax.experimental import pallas as pl
from jax.experimental.pallas import tpu as pltpu
from jax.experimental.pallas import tpu_sc as plsc

N_CORES = 2
N_SUBCORES = 16
N_TILES = N_CORES * N_SUBCORES
CHUNK = 128


def _combine_body(p0_ref, p1_ref, d0_ref, d1_ref, wt_ref, b_ref, o_ref):
    p = p0_ref[0] + p1_ref[0]
    deg = d0_ref[0, :, 0] + d1_ref[0, :, 0]
    o_ref[...] = (
        jnp.dot(p, wt_ref[...], preferred_element_type=jnp.float32)
        + deg[:, None] * b_ref[...]
    )


def _make_sc_body(n_chunks, rows_per_tile, d):
    def body(x_ref, src_ref, dst_ref, out_ref, deg_out_ref,
             src_v, dst_v, buf_a, buf_b, ones_v, zeros_v,
             acc, acc_deg, sem_a, sem_b):
        c = lax.axis_index("c")
        s = lax.axis_index("s")
        wid = c * N_SUBCORES + s
        nh = n_chunks // 2

        def zrow(r, carry):
            for cc in range(d // 16):
                buf_a[r, pl.ds(cc * 16, 16)] = jnp.zeros((16,), jnp.float32)
            return carry
        lax.fori_loop(0, CHUNK, zrow, 0)
        for cc in range(CHUNK // 16):
            ones_v[pl.ds(cc * 16, 16)] = jnp.ones((16,), jnp.float32)
        for cc in range(rows_per_tile // 16 + 1):
            zeros_v[pl.ds(cc * 16, 16)] = jnp.zeros((16,), jnp.float32)

        zbase = s * rows_per_tile
        n_full = rows_per_tile // CHUNK
        rem = rows_per_tile - n_full * CHUNK
        for j in range(n_full):
            pltpu.sync_copy(buf_a, acc.at[pl.ds(zbase + j * CHUNK, CHUNK)])
        if rem:
            pltpu.sync_copy(buf_a.at[pl.ds(0, rem)],
                            acc.at[pl.ds(zbase + n_full * CHUNK, rem)])
        pltpu.sync_copy(zeros_v.at[pl.ds(0, rows_per_tile)],
                        acc_deg.at[pl.ds(zbase, rows_per_tile)])
        plsc.subcore_barrier()

        def start_gather(i, buf, sem):
            pltpu.async_copy(x_ref.at[src_v.at[i]], buf, sem)

        def wait_gather(buf, sem):
            pltpu.make_async_copy(x_ref.at[pl.ds(0, CHUNK)], buf, sem).wait()

        def scatter(i, buf):
            pltpu.sync_copy(buf, acc.at[dst_v.at[i]], add=True)
            pltpu.sync_copy(ones_v, acc_deg.at[dst_v.at[i]], add=True)

        for half in range(2):
            pltpu.sync_copy(src_ref.at[wid, pl.ds(half * nh, nh)], src_v)
            pltpu.sync_copy(dst_ref.at[wid, pl.ds(half * nh, nh)], dst_v)

            start_gather(0, buf_a, sem_a)

            def pair_step(g, carry):
                i = 2 * g
                start_gather(i + 1, buf_b, sem_b)
                wait_gather(buf_a, sem_a)
                scatter(i, buf_a)
                start_gather(i + 2, buf_a, sem_a)
                wait_gather(buf_b, sem_b)
                scatter(i + 1, buf_b)
                return carry
            lax.fori_loop(0, nh // 2 - 1, pair_step, 0)

            i = nh - 2
            start_gather(i + 1, buf_b, sem_b)
            wait_gather(buf_a, sem_a)
            scatter(i, buf_a)
            wait_gather(buf_b, sem_b)
            scatter(i + 1, buf_b)
        plsc.subcore_barrier()

        wbase = s * rows_per_tile
        pltpu.sync_copy(acc.at[pl.ds(wbase, rows_per_tile)],
                        out_ref.at[c, pl.ds(wbase, rows_per_tile)])
        pltpu.sync_copy(acc_deg.at[pl.ds(wbase, rows_per_tile)],
                        zeros_v.at[pl.ds(0, rows_per_tile)])
        pltpu.sync_copy(
            zeros_v.at[pl.ds(0, rows_per_tile)],
            deg_out_ref.at[pl.ds(c * (N_SUBCORES * rows_per_tile) + wbase,
                                 rows_per_tile)])

    return body


@jax.jit
def kernel(feat, edge_index, W, b):
    n, d_in = feat.shape
    d_out = W.shape[0]
    e = edge_index.shape[1]

    e_per_tile = e // N_TILES
    n_chunks = -(-e_per_tile // (4 * CHUNK)) * 4
    e_pad = n_chunks * CHUNK
    pad = e_pad - e_per_tile

    src = edge_index[0].astype(jnp.int32).reshape(N_TILES, e_per_tile)
    dst = edge_index[1].astype(jnp.int32).reshape(N_TILES, e_per_tile)
    if pad:
        trash = (n + 3 * jnp.arange(N_TILES, dtype=jnp.int32)[:, None]
                 + (jnp.arange(pad, dtype=jnp.int32) % 3)[None, :])
        pad_src = ((59 * jnp.arange(N_TILES, dtype=jnp.int32)[:, None]
                    + 17 * jnp.arange(pad, dtype=jnp.int32)[None, :]) % n)
        src = jnp.concatenate([src, pad_src], axis=1)
        dst = jnp.concatenate([dst, trash], axis=1)
    src = src.reshape(N_TILES, n_chunks, CHUNK)
    dst = dst.reshape(N_TILES, n_chunks, CHUNK)

    n_acc = -(-(n + 1) // (N_SUBCORES * 8)) * N_SUBCORES * 8
    rows_per_tile = n_acc // N_SUBCORES

    sc_body = _make_sc_body(n_chunks, rows_per_tile, d_in)
    mesh = plsc.VectorSubcoreMesh(core_axis_name="c", subcore_axis_name="s")
    partials, degs = pl.kernel(
        sc_body,
        mesh=mesh,
        out_type=[
            jax.ShapeDtypeStruct((N_CORES, n_acc, d_in), jnp.float32),
            jax.ShapeDtypeStruct((N_CORES * n_acc,), jnp.float32),
        ],
        scratch_types=[
            pltpu.VMEM((n_chunks // 2, CHUNK), jnp.int32),
            pltpu.VMEM((n_chunks // 2, CHUNK), jnp.int32),
            pltpu.VMEM((CHUNK, d_in), jnp.float32),
            pltpu.VMEM((CHUNK, d_in), jnp.float32),
            pltpu.VMEM((CHUNK,), jnp.float32),
            pltpu.VMEM((rows_per_tile + 16,), jnp.float32),
            pltpu.VMEM_SHARED((n_acc, d_in), jnp.float32),
            pltpu.VMEM_SHARED((n_acc,), jnp.float32),
            pltpu.SemaphoreType.DMA,
            pltpu.SemaphoreType.DMA,
        ],
    )(feat, src, dst)

    degs3 = degs.reshape(N_CORES, n_acc, 1)
    row_blk = 1000
    out = pl.pallas_call(
        _combine_body,
        grid=(n // row_blk,),
        in_specs=[
            pl.BlockSpec((1, row_blk, d_in), lambda i: (0, i, 0)),
            pl.BlockSpec((1, row_blk, d_in), lambda i: (1, i, 0)),
            pl.BlockSpec((1, row_blk, 1), lambda i: (0, i, 0)),
            pl.BlockSpec((1, row_blk, 1), lambda i: (1, i, 0)),
            pl.BlockSpec((d_in, d_out), lambda i: (0, 0)),
            pl.BlockSpec((1, d_out), lambda i: (0, 0)),
        ],
        out_specs=pl.BlockSpec((row_blk, d_out), lambda i: (i, 0)),
        out_shape=jax.ShapeDtypeStruct((n, d_out), jnp.float32),
    )(partials, partials, degs3, degs3, W.T, b[None, :])
    return out


# speedup vs baseline: 1.0406x; 1.0406x over previous
import functools

import jax
import jax.numpy as jnp
from jax import lax
from jax.experimental import pallas as pl
from jax.experimental.pallas import tpu as pltpu
from jax.experimental.pallas import tpu_sc as plsc

N_CORES = 2
N_SUBCORES = 16
N_TILES = N_CORES * N_SUBCORES
CHUNK = 128


def _linear_body(x_ref, wt_ref, b_ref, o_ref):
    o_ref[...] = (
        jnp.dot(x_ref[...], wt_ref[...], preferred_element_type=jnp.float32)
        + b_ref[...]
    )


def _combine_body(p0_ref, p1_ref, o_ref):
    o_ref[...] = p0_ref[0] + p1_ref[0]


def _make_sc_body(n_chunks, rows_per_tile, last_rows, d):
    def body(h_ref, src_ref, dst_ref, out_ref,
             src_v, dst_v, buf_a, buf_b, acc, sem_a, sem_b):
        c = lax.axis_index("c")
        s = lax.axis_index("s")
        wid = c * N_SUBCORES + s
        nh = n_chunks // 2

        def zrow(r, carry):
            for cc in range(d // 16):
                buf_a[r, pl.ds(cc * 16, 16)] = jnp.zeros((16,), jnp.float32)
            return carry
        lax.fori_loop(0, CHUNK, zrow, 0)

        zbase = s * rows_per_tile
        n_full = rows_per_tile // CHUNK
        rem = rows_per_tile - n_full * CHUNK
        for j in range(n_full):
            pltpu.sync_copy(buf_a, acc.at[pl.ds(zbase + j * CHUNK, CHUNK)])
        if rem:
            pltpu.sync_copy(buf_a.at[pl.ds(0, rem)],
                            acc.at[pl.ds(zbase + n_full * CHUNK, rem)])
        plsc.subcore_barrier()

        def start_gather(i, buf, sem):
            pltpu.async_copy(h_ref.at[src_v.at[i]], buf, sem)

        def wait_gather(buf, sem):
            pltpu.make_async_copy(h_ref.at[pl.ds(0, CHUNK)], buf, sem).wait()

        def scatter(i, buf):
            pltpu.sync_copy(buf, acc.at[dst_v.at[i]], add=True)

        for half in range(2):
            pltpu.sync_copy(src_ref.at[wid, pl.ds(half * nh, nh)], src_v)
            pltpu.sync_copy(dst_ref.at[wid, pl.ds(half * nh, nh)], dst_v)

            start_gather(0, buf_a, sem_a)

            def pair_step(g, carry):
                i = 2 * g
                start_gather(i + 1, buf_b, sem_b)
                wait_gather(buf_a, sem_a)
                scatter(i, buf_a)
                start_gather(i + 2, buf_a, sem_a)
                wait_gather(buf_b, sem_b)
                scatter(i + 1, buf_b)
                return carry
            lax.fori_loop(0, nh // 2 - 1, pair_step, 0)

            i = nh - 2
            start_gather(i + 1, buf_b, sem_b)
            wait_gather(buf_a, sem_a)
            scatter(i, buf_a)
            wait_gather(buf_b, sem_b)
            scatter(i + 1, buf_b)
        plsc.subcore_barrier()

        wbase = s * rows_per_tile
        pltpu.sync_copy(acc.at[pl.ds(wbase, rows_per_tile)],
                        out_ref.at[c, pl.ds(wbase, rows_per_tile)])

    return body


@jax.jit
def kernel(feat, edge_index, W, b):
    n, d_in = feat.shape
    d_out = W.shape[0]
    e = edge_index.shape[1]

    row_blk = 1000
    h = pl.pallas_call(
        _linear_body,
        grid=(n // row_blk,),
        in_specs=[
            pl.BlockSpec((row_blk, d_in), lambda i: (i, 0)),
            pl.BlockSpec((d_in, d_out), lambda i: (0, 0)),
            pl.BlockSpec((1, d_out), lambda i: (0, 0)),
        ],
        out_specs=pl.BlockSpec((row_blk, d_out), lambda i: (i, 0)),
        out_shape=jax.ShapeDtypeStruct((n, d_out), jnp.float32),
    )(feat, W.T, b[None, :])

    e_per_tile = e // N_TILES
    n_chunks = -(-e_per_tile // (4 * CHUNK)) * 4
    e_pad = n_chunks * CHUNK
    pad = e_pad - e_per_tile

    src = edge_index[0].astype(jnp.int32).reshape(N_TILES, e_per_tile)
    dst = edge_index[1].astype(jnp.int32).reshape(N_TILES, e_per_tile)
    if pad:
        trash = (n + 3 * jnp.arange(N_TILES, dtype=jnp.int32)[:, None]
                 + (jnp.arange(pad, dtype=jnp.int32) % 3)[None, :])
        pad_src = ((59 * jnp.arange(N_TILES, dtype=jnp.int32)[:, None]
                    + 17 * jnp.arange(pad, dtype=jnp.int32)[None, :]) % n)
        src = jnp.concatenate([src, pad_src], axis=1)
        dst = jnp.concatenate([dst, trash], axis=1)
    src = src.reshape(N_TILES, n_chunks, CHUNK)
    dst = dst.reshape(N_TILES, n_chunks, CHUNK)

    n_acc = -(-(n + 1) // (N_SUBCORES * 8)) * N_SUBCORES * 8
    rows_per_tile = n_acc // N_SUBCORES

    sc_body = _make_sc_body(n_chunks, rows_per_tile, rows_per_tile, d_out)
    mesh = plsc.VectorSubcoreMesh(core_axis_name="c", subcore_axis_name="s")
    partials = pl.kernel(
        sc_body,
        mesh=mesh,
        out_type=jax.ShapeDtypeStruct((N_CORES, n_acc, d_out), jnp.float32),
        scratch_types=[
            pltpu.VMEM((n_chunks // 2, CHUNK), jnp.int32),
            pltpu.VMEM((n_chunks // 2, CHUNK), jnp.int32),
            pltpu.VMEM((CHUNK, d_out), jnp.float32),
            pltpu.VMEM((CHUNK, d_out), jnp.float32),
            pltpu.VMEM_SHARED((n_acc, d_out), jnp.float32),
            pltpu.SemaphoreType.DMA,
            pltpu.SemaphoreType.DMA,
        ],
    )(h, src, dst)

    out = pl.pallas_call(
        _combine_body,
        grid=(n // row_blk,),
        in_specs=[
            pl.BlockSpec((1, row_blk, d_out), lambda i: (0, i, 0)),
            pl.BlockSpec((1, row_blk, d_out), lambda i: (1, i, 0)),
        ],
        out_specs=pl.BlockSpec((row_blk, d_out), lambda i: (i, 0)),
        out_shape=jax.ShapeDtypeStruct((n, d_out), jnp.float32),
    )(partials, partials)
    return out


# speedup vs baseline: 1.0528x; 1.0117x over previous
import functools

import jax
import jax.numpy as jnp
from jax import lax
from jax.experimental import pallas as pl
from jax.experimental.pallas import tpu as pltpu
from jax.experimental.pallas import tpu_sc as plsc

N_CORES = 2
N_SUBCORES = 16
N_TILES = N_CORES * N_SUBCORES
CHUNK = 128


def _linear_body(x_ref, wt_ref, b_ref, o_ref):
    o_ref[...] = (
        jnp.dot(x_ref[...], wt_ref[...], preferred_element_type=jnp.float32)
        + b_ref[...]
    )


def _combine_body(p0_ref, p1_ref, o_ref):
    o_ref[...] = p0_ref[0] + p1_ref[0]


def _make_sc_body(n_chunks, rows_per_tile, last_rows, d):
    def body(h_ref, src_ref, dst_ref, out_ref,
             src_v, dst_v, buf_a, buf_b, acc, sem_a, sem_b):
        c = lax.axis_index("c")
        s = lax.axis_index("s")
        wid = c * N_SUBCORES + s
        nh = n_chunks // 2

        def start_gather(i, buf, sem):
            pltpu.async_copy(h_ref.at[src_v.at[i]], buf, sem)

        def wait_gather(buf, sem):
            pltpu.make_async_copy(h_ref.at[pl.ds(0, CHUNK)], buf, sem).wait()

        def scatter(i, buf):
            pltpu.sync_copy(buf, acc.at[dst_v.at[i]], add=True)

        def run_half(buf_x, sem_x, buf_y, sem_y):
            def pair_step(g, carry):
                i = 2 * g
                start_gather(i + 1, buf_y, sem_y)
                wait_gather(buf_x, sem_x)
                scatter(i, buf_x)
                start_gather(i + 2, buf_x, sem_x)
                wait_gather(buf_y, sem_y)
                scatter(i + 1, buf_y)
                return carry
            lax.fori_loop(0, nh // 2 - 1, pair_step, 0)

            i = nh - 2
            start_gather(i + 1, buf_y, sem_y)
            wait_gather(buf_x, sem_x)
            scatter(i, buf_x)
            wait_gather(buf_y, sem_y)
            scatter(i + 1, buf_y)

        pltpu.sync_copy(src_ref.at[wid, pl.ds(0, nh)], src_v)
        pltpu.sync_copy(dst_ref.at[wid, pl.ds(0, nh)], dst_v)
        start_gather(0, buf_b, sem_b)

        def zrow(r, carry):
            for cc in range(d // 16):
                buf_a[r, pl.ds(cc * 16, 16)] = jnp.zeros((16,), jnp.float32)
            return carry
        lax.fori_loop(0, CHUNK, zrow, 0)

        zbase = s * rows_per_tile
        n_full = rows_per_tile // CHUNK
        rem = rows_per_tile - n_full * CHUNK
        for j in range(n_full):
            pltpu.sync_copy(buf_a, acc.at[pl.ds(zbase + j * CHUNK, CHUNK)])
        if rem:
            pltpu.sync_copy(buf_a.at[pl.ds(0, rem)],
                            acc.at[pl.ds(zbase + n_full * CHUNK, rem)])
        plsc.subcore_barrier()

        run_half(buf_b, sem_b, buf_a, sem_a)

        pltpu.sync_copy(src_ref.at[wid, pl.ds(nh, nh)], src_v)
        pltpu.sync_copy(dst_ref.at[wid, pl.ds(nh, nh)], dst_v)
        start_gather(0, buf_a, sem_a)
        run_half(buf_a, sem_a, buf_b, sem_b)
        plsc.subcore_barrier()

        wbase = s * rows_per_tile
        pltpu.sync_copy(acc.at[pl.ds(wbase, rows_per_tile)],
                        out_ref.at[c, pl.ds(wbase, rows_per_tile)])

    return body


@jax.jit
def kernel(feat, edge_index, W, b):
    n, d_in = feat.shape
    d_out = W.shape[0]
    e = edge_index.shape[1]

    row_blk = 1000
    h = pl.pallas_call(
        _linear_body,
        grid=(n // row_blk,),
        in_specs=[
            pl.BlockSpec((row_blk, d_in), lambda i: (i, 0)),
            pl.BlockSpec((d_in, d_out), lambda i: (0, 0)),
            pl.BlockSpec((1, d_out), lambda i: (0, 0)),
        ],
        out_specs=pl.BlockSpec((row_blk, d_out), lambda i: (i, 0)),
        out_shape=jax.ShapeDtypeStruct((n, d_out), jnp.float32),
    )(feat, W.T, b[None, :])

    e_per_tile = e // N_TILES
    n_chunks = -(-e_per_tile // (4 * CHUNK)) * 4
    e_pad = n_chunks * CHUNK
    pad = e_pad - e_per_tile

    src = edge_index[0].astype(jnp.int32).reshape(N_TILES, e_per_tile)
    dst = edge_index[1].astype(jnp.int32).reshape(N_TILES, e_per_tile)
    if pad:
        trash = (n + 3 * jnp.arange(N_TILES, dtype=jnp.int32)[:, None]
                 + (jnp.arange(pad, dtype=jnp.int32) % 3)[None, :])
        pad_src = ((59 * jnp.arange(N_TILES, dtype=jnp.int32)[:, None]
                    + 17 * jnp.arange(pad, dtype=jnp.int32)[None, :]) % n)
        src = jnp.concatenate([src, pad_src], axis=1)
        dst = jnp.concatenate([dst, trash], axis=1)
    src = src.reshape(N_TILES, n_chunks, CHUNK)
    dst = dst.reshape(N_TILES, n_chunks, CHUNK)

    n_acc = -(-(n + 1) // (N_SUBCORES * 8)) * N_SUBCORES * 8
    rows_per_tile = n_acc // N_SUBCORES

    sc_body = _make_sc_body(n_chunks, rows_per_tile, rows_per_tile, d_out)
    mesh = plsc.VectorSubcoreMesh(core_axis_name="c", subcore_axis_name="s")
    partials = pl.kernel(
        sc_body,
        mesh=mesh,
        out_type=jax.ShapeDtypeStruct((N_CORES, n_acc, d_out), jnp.float32),
        scratch_types=[
            pltpu.VMEM((n_chunks // 2, CHUNK), jnp.int32),
            pltpu.VMEM((n_chunks // 2, CHUNK), jnp.int32),
            pltpu.VMEM((CHUNK, d_out), jnp.float32),
            pltpu.VMEM((CHUNK, d_out), jnp.float32),
            pltpu.VMEM_SHARED((n_acc, d_out), jnp.float32),
            pltpu.SemaphoreType.DMA,
            pltpu.SemaphoreType.DMA,
        ],
    )(h, src, dst)

    out = pl.pallas_call(
        _combine_body,
        grid=(n // row_blk,),
        in_specs=[
            pl.BlockSpec((1, row_blk, d_out), lambda i: (0, i, 0)),
            pl.BlockSpec((1, row_blk, d_out), lambda i: (1, i, 0)),
        ],
        out_specs=pl.BlockSpec((row_blk, d_out), lambda i: (i, 0)),
        out_shape=jax.ShapeDtypeStruct((n, d_out), jnp.float32),
    )(partials, partials)
    return out
